# baseline (speedup 1.0000x reference)
import functools

import jax
import jax.numpy as jnp
from jax import lax
from jax.experimental import pallas as pl
from jax.experimental.pallas import tpu as pltpu
from jax.experimental.pallas import tpu_sc as plsc

N_NODE = 50000
D_IN = 128
NGRP = 4
GW = 32
EP = 303104
NROW_E = EP // 128
NBLK_F = NROW_E // 16
NBLK_D = NROW_E // 32
NP = 50176
RPT = NP // 16
ZCH = 112


def _sc_body(x4s, src4, dstp, acc_out, deg_out,
             sidx_v, didx_v, rows_v, zbuf, zbuff, obuf, acc_sh, deg_sh,
             si0, si1, si2, si3, sg0, sg1, sg2, sg3):
    c = lax.axis_index("c")
    s = lax.axis_index("s")
    row0 = s * RPT
    sis = (si0, si1, si2, si3)
    sgs = (sg0, sg1, sg2, sg3)

    z32 = jnp.zeros((32,), jnp.bfloat16)
    zf16 = jnp.zeros((16,), jnp.float32)
    o16 = jnp.ones((16,), jnp.float32)

    def fill_z(r, _):
        zbuf[r, 0:32] = z32
        zbuff[r, 0:16] = zf16
        return 0
    lax.fori_loop(0, ZCH, fill_z, 0)

    def fill_o(r, _):
        obuf[r, 0:16] = o16
        return 0
    lax.fori_loop(0, 128, fill_o, 0)

    def zero_acc():
        def zb(i, _):
            pltpu.sync_copy(zbuf, acc_sh.at[pl.ds(row0 + i * ZCH, ZCH)])
            return 0
        lax.fori_loop(0, RPT // ZCH, zb, 0)

    def zero_deg():
        def zb(i, _):
            pltpu.sync_copy(zbuff, deg_sh.at[pl.ds(row0 + i * ZCH, ZCH)])
            return 0
        lax.fori_loop(0, RPT // ZCH, zb, 0)

    def gstart(j, rs, sem):
        pltpu.async_copy(x4s.at[sidx_v.at[j]], rows_v.at[rs], sem)

    def gwait(j, rs, sem):
        pltpu.make_async_copy(x4s.at[sidx_v.at[j]], rows_v.at[rs],
                              sem).wait()

    def scat(j, rs):
        pltpu.sync_copy(rows_v.at[rs], acc_sh.at[didx_v.at[j]], add=True)

    def feature_pass(gg):
        g = c * 2 + gg
        zero_acc()
        plsc.subcore_barrier()
        brow = s * NBLK_F
        ngroups = NBLK_F // 4

        def idx_start(k, half, j):
            r = brow + 4 * k + j
            pltpu.async_copy(src4.at[g, r], sidx_v.at[half + j], sis[j])
            pltpu.async_copy(dstp.at[r], didx_v.at[half + j], sis[j])

        def idx_wait(k, half, j):
            r = brow + 4 * k + j
            pltpu.make_async_copy(src4.at[g, r], sidx_v.at[half + j],
                                  sis[j]).wait()
            pltpu.make_async_copy(dstp.at[r], didx_v.at[half + j],
                                  sis[j]).wait()

        for j in range(4):
            idx_start(0, 0, j)

        def body(k, _):
            half = (k % 2) * 4
            nhalf = 4 - half
            for j in range(4):
                idx_wait(k, half, j)
                gstart(half + j, j, sgs[j])

            @pl.when(k + 1 < ngroups)
            def _():
                for j in range(4):
                    idx_start(k + 1, nhalf, j)

            for j in range(4):
                gwait(half + j, j, sgs[j])
                scat(half + j, j)
            return 0
        lax.fori_loop(0, ngroups, body, 0)
        plsc.subcore_barrier()
        pltpu.sync_copy(acc_sh.at[pl.ds(row0, RPT)],
                        acc_out.at[g, pl.ds(row0, RPT)])
        plsc.subcore_barrier()

    feature_pass(0)
    feature_pass(1)

    zero_deg()
    plsc.subcore_barrier()
    drow = c * (NROW_E // 2) + s * NBLK_D

    def dstart(k, j):
        pltpu.async_copy(dstp.at[drow + 2 * k + j], didx_v.at[j], sis[j])

    def dwait(k, j):
        pltpu.make_async_copy(dstp.at[drow + 2 * k + j], didx_v.at[j],
                              sis[j]).wait()

    def dblk(k, _):
        dstart(k, 0)
        dstart(k, 1)
        dwait(k, 0)
        pltpu.sync_copy(obuf, deg_sh.at[didx_v.at[0]], add=True)
        dwait(k, 1)
        pltpu.sync_copy(obuf, deg_sh.at[didx_v.at[1]], add=True)
        return 0
    lax.fori_loop(0, NBLK_D // 2, dblk, 0)
    plsc.subcore_barrier()
    pltpu.sync_copy(deg_sh.at[pl.ds(row0, RPT)],
                    deg_out.at[c, pl.ds(row0, RPT)])


@functools.cache
def _sc_scatter():
  return pl.kernel(
    _sc_body,
    out_type=[
        jax.ShapeDtypeStruct((NGRP, NP, GW), jnp.bfloat16),
        jax.ShapeDtypeStruct((2, NP, 16), jnp.float32),
    ],
    mesh=plsc.VectorSubcoreMesh(core_axis_name="c", subcore_axis_name="s",
                                num_cores=2, num_subcores=16),
    scratch_types=[
        pltpu.VMEM((8, 128), jnp.int32),
        pltpu.VMEM((8, 128), jnp.int32),
        pltpu.VMEM((4, 128, GW), jnp.bfloat16),
        pltpu.VMEM((ZCH, GW), jnp.bfloat16),
        pltpu.VMEM((ZCH, 16), jnp.float32),
        pltpu.VMEM((128, 16), jnp.float32),
        pltpu.VMEM_SHARED((NP, GW), jnp.bfloat16),
        pltpu.VMEM_SHARED((NP, 16), jnp.float32),
        pltpu.SemaphoreType.DMA,
        pltpu.SemaphoreType.DMA,
        pltpu.SemaphoreType.DMA,
        pltpu.SemaphoreType.DMA,
        pltpu.SemaphoreType.DMA,
        pltpu.SemaphoreType.DMA,
        pltpu.SemaphoreType.DMA,
        pltpu.SemaphoreType.DMA,
    ],
    compiler_params=pltpu.CompilerParams(use_tc_tiling_on_sc=False),
  )


BLK = 512
_GRID = NP // BLK


def _tc_body(x_ref, acc_ref, deg_ref,
             wself_ref, bself_ref, wconv_ref, bconv_ref,
             wq_ref, bq_ref, wk_ref, bk_ref,
             wal_ref, bal_ref, war_ref, bar_ref,
             wcls_ref, bcls_ref, out_ref):
    f32 = jnp.float32
    x = x_ref[...]
    z = jnp.dot(x, wself_ref[...], preferred_element_type=f32) + bself_ref[...]
    wconv = wconv_ref[...]
    acc = acc_ref[...].astype(f32)
    t = jnp.dot(acc[0], wconv[0:GW, :], preferred_element_type=f32)
    for g in range(1, NGRP):
        t = t + jnp.dot(acc[g], wconv[g * GW:(g + 1) * GW, :],
                        preferred_element_type=f32)
    degf = deg_ref[0] + deg_ref[1]
    deg1 = jnp.sum(degf, axis=1, keepdims=True) * (1.0 / 16.0)
    rdeg = 1.0 / jnp.maximum(deg1, 1.0)
    d = t * rdeg + bconv_ref[...]
    wal = wal_ref[...]
    war = war_ref[...]
    wkal = jnp.sum(wk_ref[...] * wal, axis=1)
    wqar = jnp.sum(wq_ref[...] * war, axis=1)
    cl = jnp.sum(bk_ref[...] * wal, axis=1, keepdims=True) + bal_ref[...]
    cr = jnp.sum(bq_ref[...] * war, axis=1, keepdims=True) + bar_ref[...]
    hl = jnp.sum(z * wkal[None, :], axis=1, keepdims=True) + cl
    hr = jnp.sum(z * wqar[None, :], axis=1, keepdims=True) + cr
    ha = jnp.sum(d * wkal[None, :], axis=1, keepdims=True) + cl

    def elu(v):
        return jnp.where(v > 0, v, jnp.exp(v) - 1.0)

    a0 = elu(hl + hr)
    a1 = elu(ha + hr)
    mx = jnp.maximum(a0, a1)
    e0 = jnp.exp(a0 - mx)
    e1 = jnp.exp(a1 - mx)
    inv = 1.0 / (e0 + e1)
    rst = z * (e0 * inv) + d * (e1 * inv)
    out_ref[...] = (jnp.dot(rst, wcls_ref[...], preferred_element_type=f32)
                    + bcls_ref[...])


def _full(shape):
    return pl.BlockSpec(shape, lambda i: (0,) * len(shape))


_tc_epilogue = pl.pallas_call(
    _tc_body,
    grid=(_GRID,),
    in_specs=[
        pl.BlockSpec((BLK, D_IN), lambda i: (i, 0)),
        pl.BlockSpec((NGRP, BLK, GW), lambda i: (0, i, 0)),
        pl.BlockSpec((2, BLK, 16), lambda i: (0, i, 0)),
        _full((D_IN, D_IN)), _full((1, D_IN)),
        _full((D_IN, D_IN)), _full((1, D_IN)),
        _full((D_IN, D_IN)), _full((1, D_IN)),
        _full((D_IN, D_IN)), _full((1, D_IN)),
        _full((1, D_IN)), _full((1, 1)),
        _full((1, D_IN)), _full((1, 1)),
        _full((D_IN, 16)), _full((1, 16)),
    ],
    out_specs=pl.BlockSpec((BLK, 16), lambda i: (i, 0)),
    out_shape=jax.ShapeDtypeStruct((N_NODE, 16), jnp.float32),
)


def kernel(x_paper, x_author, edge_index_writes, edge_index_written_by,
           Wself_paper, bself_paper, Wself_author, bself_author,
           Wq_paper, bq_paper, Wk_paper, bk_paper,
           Wq_author, bq_author, Wk_author, bk_author,
           Wal_paper, bal_paper, Wal_author, bal_author,
           War_paper, bar_paper, War_author, bar_author,
           Wconv_writes, bconv_writes, Wconv_written_by, bconv_written_by,
           Wcls, bcls):
    e = edge_index_writes.shape[1]
    src = edge_index_writes[0]
    dst = edge_index_writes[1]
    npad = EP - e
    pad_src = jnp.arange(npad, dtype=jnp.int32) % N_NODE
    pad_dst = N_NODE + jnp.arange(npad, dtype=jnp.int32) % (NP - N_NODE)
    srcp = jnp.concatenate([src.astype(jnp.int32), pad_src])
    dstp = jnp.concatenate([dst.astype(jnp.int32), pad_dst])
    src4 = (srcp[None, :] * NGRP
            + jnp.arange(NGRP, dtype=jnp.int32)[:, None]).reshape(
                NGRP, NROW_E, 128)
    x4s = x_author.astype(jnp.bfloat16).reshape(NGRP * N_NODE, GW)

    acc4, deg2 = _sc_scatter()(x4s, src4, dstp.reshape(NROW_E, 128))

    logits = _tc_epilogue(
        x_paper, acc4, deg2,
        Wself_paper, bself_paper.reshape(1, D_IN),
        Wconv_writes, bconv_writes.reshape(1, D_IN),
        Wq_paper, bq_paper.reshape(1, D_IN),
        Wk_paper, bk_paper.reshape(1, D_IN),
        Wal_paper.reshape(1, D_IN), bal_paper.reshape(1, 1),
        War_paper.reshape(1, D_IN), bar_paper.reshape(1, 1),
        Wcls, bcls.reshape(1, 16),
    )
    return logits


# speedup vs baseline: 1.2124x; 1.2124x over previous
import functools

import jax
import jax.numpy as jnp
from jax import lax
from jax.experimental import pallas as pl
from jax.experimental.pallas import tpu as pltpu
from jax.experimental.pallas import tpu_sc as plsc

N_NODE = 50000
D_IN = 128
NGRP = 2
GW = 64
EP = 303104
NROW_E = EP // 128
NBLK_F = NROW_E // 16
NBLK_D = NROW_E // 32
NP = 50176
RPT = NP // 16
ZCH = 56
ZCHF = 112


def _sc_feat_body(x2s, src2, dstp, acc_out,
                  sidx_v, didx_v, rows_v, zbuf, acc_sh,
                  si0, si1, si2, si3, sg0, sg1, sg2, sg3):
    c = lax.axis_index("c")
    s = lax.axis_index("s")
    row0 = s * RPT
    sis = (si0, si1, si2, si3)
    sgs = (sg0, sg1, sg2, sg3)

    z32 = jnp.zeros((32,), jnp.bfloat16)

    def fill_z(r, _):
        zbuf[r, 0:32] = z32
        zbuf[r, 32:64] = z32
        return 0
    lax.fori_loop(0, ZCH, fill_z, 0)

    def zb(i, _):
        pltpu.sync_copy(zbuf, acc_sh.at[pl.ds(row0 + i * ZCH, ZCH)])
        return 0
    lax.fori_loop(0, RPT // ZCH, zb, 0)
    plsc.subcore_barrier()

    brow = s * NBLK_F
    ngroups = NBLK_F // 4

    def idx_start(k, half, j):
        r = brow + 4 * k + j
        pltpu.async_copy(src2.at[c, r], sidx_v.at[half + j], sis[j])
        pltpu.async_copy(dstp.at[r], didx_v.at[half + j], sis[j])

    def idx_wait(k, half, j):
        r = brow + 4 * k + j
        pltpu.make_async_copy(src2.at[c, r], sidx_v.at[half + j],
                              sis[j]).wait()
        pltpu.make_async_copy(dstp.at[r], didx_v.at[half + j],
                              sis[j]).wait()

    def gstart(j, rs, sem):
        pltpu.async_copy(x2s.at[sidx_v.at[j]], rows_v.at[rs], sem)

    def gwait(j, rs, sem):
        pltpu.make_async_copy(x2s.at[sidx_v.at[j]], rows_v.at[rs],
                              sem).wait()

    def scat(j, rs):
        pltpu.sync_copy(rows_v.at[rs], acc_sh.at[didx_v.at[j]], add=True)

    for j in range(4):
        idx_start(0, 0, j)

    def body(k, _):
        half = (k % 2) * 4
        nhalf = 4 - half
        for j in range(4):
            idx_wait(k, half, j)
            gstart(half + j, j, sgs[j])

        @pl.when(k + 1 < ngroups)
        def _():
            for j in range(4):
                idx_start(k + 1, nhalf, j)

        for j in range(4):
            gwait(half + j, j, sgs[j])
            scat(half + j, j)
        return 0
    lax.fori_loop(0, ngroups, body, 0)
    plsc.subcore_barrier()
    pltpu.sync_copy(acc_sh.at[pl.ds(row0, RPT)],
                    acc_out.at[c, pl.ds(row0, RPT)])


def _sc_deg_body(dstp, deg_out, didx_v, zbuff, obuf, deg_sh, si0, si1):
    c = lax.axis_index("c")
    s = lax.axis_index("s")
    row0 = s * RPT
    sis = (si0, si1)

    zf16 = jnp.zeros((16,), jnp.float32)
    o16 = jnp.ones((16,), jnp.float32)

    def fill_zf(r, _):
        zbuff[r, 0:16] = zf16
        return 0
    lax.fori_loop(0, ZCHF, fill_zf, 0)

    def fill_o(r, _):
        obuf[r, 0:16] = o16
        return 0
    lax.fori_loop(0, 128, fill_o, 0)

    def zb(i, _):
        pltpu.sync_copy(zbuff, deg_sh.at[pl.ds(row0 + i * ZCHF, ZCHF)])
        return 0
    lax.fori_loop(0, RPT // ZCHF, zb, 0)
    plsc.subcore_barrier()
    drow = c * (NROW_E // 2) + s * NBLK_D

    def dstart(k, j):
        pltpu.async_copy(dstp.at[drow + 2 * k + j], didx_v.at[j], sis[j])

    def dwait(k, j):
        pltpu.make_async_copy(dstp.at[drow + 2 * k + j], didx_v.at[j],
                              sis[j]).wait()

    def dblk(k, _):
        dstart(k, 0)
        dstart(k, 1)
        dwait(k, 0)
        pltpu.sync_copy(obuf, deg_sh.at[didx_v.at[0]], add=True)
        dwait(k, 1)
        pltpu.sync_copy(obuf, deg_sh.at[didx_v.at[1]], add=True)
        return 0
    lax.fori_loop(0, NBLK_D // 2, dblk, 0)
    plsc.subcore_barrier()
    pltpu.sync_copy(deg_sh.at[pl.ds(row0, RPT)],
                    deg_out.at[c, pl.ds(row0, RPT)])


@functools.cache
def _sc_scatter():
  return pl.kernel(
    _sc_feat_body,
    out_type=jax.ShapeDtypeStruct((NGRP, NP, GW), jnp.bfloat16),
    mesh=plsc.VectorSubcoreMesh(core_axis_name="c", subcore_axis_name="s",
                                num_cores=2, num_subcores=16),
    scratch_types=[
        pltpu.VMEM((8, 128), jnp.int32),
        pltpu.VMEM((8, 128), jnp.int32),
        pltpu.VMEM((4, 128, GW), jnp.bfloat16),
        pltpu.VMEM((ZCH, GW), jnp.bfloat16),
        pltpu.VMEM_SHARED((NP, GW), jnp.bfloat16),
        pltpu.SemaphoreType.DMA,
        pltpu.SemaphoreType.DMA,
        pltpu.SemaphoreType.DMA,
        pltpu.SemaphoreType.DMA,
        pltpu.SemaphoreType.DMA,
        pltpu.SemaphoreType.DMA,
        pltpu.SemaphoreType.DMA,
        pltpu.SemaphoreType.DMA,
    ],
    compiler_params=pltpu.CompilerParams(use_tc_tiling_on_sc=False),
  )


@functools.cache
def _sc_degree():
  return pl.kernel(
    _sc_deg_body,
    out_type=jax.ShapeDtypeStruct((2, NP, 16), jnp.float32),
    mesh=plsc.VectorSubcoreMesh(core_axis_name="c", subcore_axis_name="s",
                                num_cores=2, num_subcores=16),
    scratch_types=[
        pltpu.VMEM((2, 128), jnp.int32),
        pltpu.VMEM((ZCHF, 16), jnp.float32),
        pltpu.VMEM((128, 16), jnp.float32),
        pltpu.VMEM_SHARED((NP, 16), jnp.float32),
        pltpu.SemaphoreType.DMA,
        pltpu.SemaphoreType.DMA,
    ],
    compiler_params=pltpu.CompilerParams(use_tc_tiling_on_sc=False),
  )


BLK = 512
_GRID = NP // BLK


def _tc_body(x_ref, acc_ref, deg_ref,
             wself_ref, bself_ref, wconv_ref, bconv_ref,
             wq_ref, bq_ref, wk_ref, bk_ref,
             wal_ref, bal_ref, war_ref, bar_ref,
             wcls_ref, bcls_ref, out_ref):
    f32 = jnp.float32
    x = x_ref[...]
    z = jnp.dot(x, wself_ref[...], preferred_element_type=f32) + bself_ref[...]
    wconv = wconv_ref[...]
    acc = acc_ref[...].astype(f32)
    t = jnp.dot(acc[0], wconv[0:GW, :], preferred_element_type=f32)
    for g in range(1, NGRP):
        t = t + jnp.dot(acc[g], wconv[g * GW:(g + 1) * GW, :],
                        preferred_element_type=f32)
    degf = deg_ref[0] + deg_ref[1]
    deg1 = jnp.sum(degf, axis=1, keepdims=True) * (1.0 / 16.0)
    rdeg = 1.0 / jnp.maximum(deg1, 1.0)
    d = t * rdeg + bconv_ref[...]
    wal = wal_ref[...]
    war = war_ref[...]
    wkal = jnp.sum(wk_ref[...] * wal, axis=1)
    wqar = jnp.sum(wq_ref[...] * war, axis=1)
    cl = jnp.sum(bk_ref[...] * wal, axis=1, keepdims=True) + bal_ref[...]
    cr = jnp.sum(bq_ref[...] * war, axis=1, keepdims=True) + bar_ref[...]
    hl = jnp.sum(z * wkal[None, :], axis=1, keepdims=True) + cl
    hr = jnp.sum(z * wqar[None, :], axis=1, keepdims=True) + cr
    ha = jnp.sum(d * wkal[None, :], axis=1, keepdims=True) + cl

    def elu(v):
        return jnp.where(v > 0, v, jnp.exp(v) - 1.0)

    a0 = elu(hl + hr)
    a1 = elu(ha + hr)
    mx = jnp.maximum(a0, a1)
    e0 = jnp.exp(a0 - mx)
    e1 = jnp.exp(a1 - mx)
    inv = 1.0 / (e0 + e1)
    rst = z * (e0 * inv) + d * (e1 * inv)
    out_ref[...] = (jnp.dot(rst, wcls_ref[...], preferred_element_type=f32)
                    + bcls_ref[...])


def _full(shape):
    return pl.BlockSpec(shape, lambda i: (0,) * len(shape))


_tc_epilogue = pl.pallas_call(
    _tc_body,
    grid=(_GRID,),
    in_specs=[
        pl.BlockSpec((BLK, D_IN), lambda i: (i, 0)),
        pl.BlockSpec((NGRP, BLK, GW), lambda i: (0, i, 0)),
        pl.BlockSpec((2, BLK, 16), lambda i: (0, i, 0)),
        _full((D_IN, D_IN)), _full((1, D_IN)),
        _full((D_IN, D_IN)), _full((1, D_IN)),
        _full((D_IN, D_IN)), _full((1, D_IN)),
        _full((D_IN, D_IN)), _full((1, D_IN)),
        _full((1, D_IN)), _full((1, 1)),
        _full((1, D_IN)), _full((1, 1)),
        _full((D_IN, 16)), _full((1, 16)),
    ],
    out_specs=pl.BlockSpec((BLK, 16), lambda i: (i, 0)),
    out_shape=jax.ShapeDtypeStruct((N_NODE, 16), jnp.float32),
)


def kernel(x_paper, x_author, edge_index_writes, edge_index_written_by,
           Wself_paper, bself_paper, Wself_author, bself_author,
           Wq_paper, bq_paper, Wk_paper, bk_paper,
           Wq_author, bq_author, Wk_author, bk_author,
           Wal_paper, bal_paper, Wal_author, bal_author,
           War_paper, bar_paper, War_author, bar_author,
           Wconv_writes, bconv_writes, Wconv_written_by, bconv_written_by,
           Wcls, bcls):
    e = edge_index_writes.shape[1]
    src = edge_index_writes[0]
    dst = edge_index_writes[1]
    npad = EP - e
    pad_src = jnp.arange(npad, dtype=jnp.int32) % N_NODE
    pad_dst = N_NODE + jnp.arange(npad, dtype=jnp.int32) % (NP - N_NODE)
    srcp = jnp.concatenate([src.astype(jnp.int32), pad_src])
    dstp = jnp.concatenate([dst.astype(jnp.int32), pad_dst])
    src2 = (srcp[None, :] * NGRP
            + jnp.arange(NGRP, dtype=jnp.int32)[:, None]).reshape(
                NGRP, NROW_E, 128)
    x2s = x_author.astype(jnp.bfloat16).reshape(NGRP * N_NODE, GW)

    dstr = dstp.reshape(NROW_E, 128)
    acc2 = _sc_scatter()(x2s, src2, dstr)
    deg2 = _sc_degree()(dstr)

    logits = _tc_epilogue(
        x_paper, acc2, deg2,
        Wself_paper, bself_paper.reshape(1, D_IN),
        Wconv_writes, bconv_writes.reshape(1, D_IN),
        Wq_paper, bq_paper.reshape(1, D_IN),
        Wk_paper, bk_paper.reshape(1, D_IN),
        Wal_paper.reshape(1, D_IN), bal_paper.reshape(1, 1),
        War_paper.reshape(1, D_IN), bar_paper.reshape(1, 1),
        Wcls, bcls.reshape(1, 16),
    )
    return logits


# speedup vs baseline: 1.3256x; 1.0934x over previous
import functools

import jax
import jax.numpy as jnp
from jax import lax
from jax.experimental import pallas as pl
from jax.experimental.pallas import tpu as pltpu
from jax.experimental.pallas import tpu_sc as plsc

N_NODE = 50000
D_IN = 128
NGRP = 2
GW = 64
EP = 303104
NROW_E = EP // 128
NBLK_F = NROW_E // 16
NBLK_D = NROW_E // 32
NP = 50176
RPT = NP // 16
ZCH = 56
ZCHF = 112


def _sc_feat_body(x2s, src2, dstp, acc_out,
                  sidx_v, didx_v, rows_v, zbuf, acc_sh,
                  si0, si1, si2, si3, sg0, sg1, sg2, sg3):
    c = lax.axis_index("c")
    s = lax.axis_index("s")
    row0 = s * RPT
    sis = (si0, si1, si2, si3)
    sgs = (sg0, sg1, sg2, sg3)

    z32 = jnp.zeros((32,), jnp.bfloat16)

    def fill_z(r, _):
        zbuf[r, 0:32] = z32
        zbuf[r, 32:64] = z32
        return 0
    lax.fori_loop(0, ZCH, fill_z, 0)

    def zb(i, _):
        pltpu.sync_copy(zbuf, acc_sh.at[pl.ds(row0 + i * ZCH, ZCH)])
        return 0
    lax.fori_loop(0, RPT // ZCH, zb, 0)
    plsc.subcore_barrier()

    brow = s * NBLK_F
    ngroups = NBLK_F // 4

    def idx_start(k, half, j):
        r = brow + 4 * k + j
        pltpu.async_copy(src2.at[c, r], sidx_v.at[half + j], sis[j])
        pltpu.async_copy(dstp.at[r], didx_v.at[half + j], sis[j])

    def idx_wait(k, half, j):
        r = brow + 4 * k + j
        pltpu.make_async_copy(src2.at[c, r], sidx_v.at[half + j],
                              sis[j]).wait()
        pltpu.make_async_copy(dstp.at[r], didx_v.at[half + j],
                              sis[j]).wait()

    def gstart(j, rs, sem):
        pltpu.async_copy(x2s.at[sidx_v.at[j]], rows_v.at[rs], sem)

    def gwait(j, rs, sem):
        pltpu.make_async_copy(x2s.at[sidx_v.at[j]], rows_v.at[rs],
                              sem).wait()

    def scat(j, rs):
        pltpu.sync_copy(rows_v.at[rs], acc_sh.at[didx_v.at[j]], add=True)

    for j in range(4):
        idx_start(0, 0, j)

    def body(k, _):
        half = (k % 2) * 4
        nhalf = 4 - half
        for j in range(4):
            idx_wait(k, half, j)
            gstart(half + j, j, sgs[j])

        @pl.when(k + 1 < ngroups)
        def _():
            for j in range(4):
                idx_start(k + 1, nhalf, j)

        for j in range(4):
            gwait(half + j, j, sgs[j])
            scat(half + j, j)
        return 0
    lax.fori_loop(0, ngroups, body, 0)
    plsc.subcore_barrier()
    pltpu.sync_copy(acc_sh.at[pl.ds(row0, RPT)],
                    acc_out.at[c, pl.ds(row0, RPT)])


def _sc_deg_body(dstp, deg_out, didx_v, zbuff, obuf, deg_sh, si0, si1):
    c = lax.axis_index("c")
    s = lax.axis_index("s")
    row0 = s * RPT
    sis = (si0, si1)

    zf16 = jnp.zeros((16,), jnp.float32)
    o16 = jnp.ones((16,), jnp.float32)

    def fill_zf(r, _):
        zbuff[r, 0:16] = zf16
        return 0
    lax.fori_loop(0, ZCHF, fill_zf, 0)

    def fill_o(r, _):
        obuf[r, 0:16] = o16
        return 0
    lax.fori_loop(0, 128, fill_o, 0)

    def zb(i, _):
        pltpu.sync_copy(zbuff, deg_sh.at[pl.ds(row0 + i * ZCHF, ZCHF)])
        return 0
    lax.fori_loop(0, RPT // ZCHF, zb, 0)
    plsc.subcore_barrier()
    drow = c * (NROW_E // 2) + s * NBLK_D

    def dstart(k, j):
        pltpu.async_copy(dstp.at[drow + 2 * k + j], didx_v.at[j], sis[j])

    def dwait(k, j):
        pltpu.make_async_copy(dstp.at[drow + 2 * k + j], didx_v.at[j],
                              sis[j]).wait()

    def dblk(k, _):
        dstart(k, 0)
        dstart(k, 1)
        dwait(k, 0)
        pltpu.sync_copy(obuf, deg_sh.at[didx_v.at[0]], add=True)
        dwait(k, 1)
        pltpu.sync_copy(obuf, deg_sh.at[didx_v.at[1]], add=True)
        return 0
    lax.fori_loop(0, NBLK_D // 2, dblk, 0)
    plsc.subcore_barrier()
    pltpu.sync_copy(deg_sh.at[pl.ds(row0, RPT)],
                    deg_out.at[c, pl.ds(row0, RPT)])


@functools.cache
def _sc_scatter():
  return pl.kernel(
    _sc_feat_body,
    out_type=jax.ShapeDtypeStruct((NGRP, NP, GW), jnp.bfloat16),
    mesh=plsc.VectorSubcoreMesh(core_axis_name="c", subcore_axis_name="s",
                                num_cores=2, num_subcores=16),
    scratch_types=[
        pltpu.VMEM((8, 128), jnp.int32),
        pltpu.VMEM((8, 128), jnp.int32),
        pltpu.VMEM((4, 128, GW), jnp.bfloat16),
        pltpu.VMEM((ZCH, GW), jnp.bfloat16),
        pltpu.VMEM_SHARED((NP, GW), jnp.bfloat16),
        pltpu.SemaphoreType.DMA,
        pltpu.SemaphoreType.DMA,
        pltpu.SemaphoreType.DMA,
        pltpu.SemaphoreType.DMA,
        pltpu.SemaphoreType.DMA,
        pltpu.SemaphoreType.DMA,
        pltpu.SemaphoreType.DMA,
        pltpu.SemaphoreType.DMA,
    ],
    compiler_params=pltpu.CompilerParams(use_tc_tiling_on_sc=False),
  )


@functools.cache
def _sc_degree():
  return pl.kernel(
    _sc_deg_body,
    out_type=jax.ShapeDtypeStruct((2, NP, 16), jnp.float32),
    mesh=plsc.VectorSubcoreMesh(core_axis_name="c", subcore_axis_name="s",
                                num_cores=2, num_subcores=16),
    scratch_types=[
        pltpu.VMEM((2, 128), jnp.int32),
        pltpu.VMEM((ZCHF, 16), jnp.float32),
        pltpu.VMEM((128, 16), jnp.float32),
        pltpu.VMEM_SHARED((NP, 16), jnp.float32),
        pltpu.SemaphoreType.DMA,
        pltpu.SemaphoreType.DMA,
    ],
    compiler_params=pltpu.CompilerParams(use_tc_tiling_on_sc=False),
  )


BLK = 1024
_GRID = NP // BLK
DROW = BLK // 128


def _tc_body(x_ref, acc_ref, deg_ref,
             wself_ref, bself_ref, wconv_ref, bconv_ref,
             wq_ref, bq_ref, wk_ref, bk_ref,
             wal_ref, bal_ref, war_ref, bar_ref,
             wcls_ref, bcls_ref, out_ref):
    f32 = jnp.float32
    x = x_ref[...]
    z = jnp.dot(x, wself_ref[...], preferred_element_type=f32) + bself_ref[...]
    wconv = wconv_ref[...]
    acc = acc_ref[...].astype(f32)
    t = jnp.dot(acc[0], wconv[0:GW, :], preferred_element_type=f32)
    for g in range(1, NGRP):
        t = t + jnp.dot(acc[g], wconv[g * GW:(g + 1) * GW, :],
                        preferred_element_type=f32)
    degf = deg_ref[0] + deg_ref[1]
    deg1 = jnp.sum(degf, axis=1, keepdims=True) * (1.0 / 16.0)
    rdeg = 1.0 / jnp.maximum(deg1, 1.0)
    d = t * rdeg + bconv_ref[...]
    wal = wal_ref[...]
    war = war_ref[...]
    wkal = jnp.sum(wk_ref[...] * wal, axis=1)
    wqar = jnp.sum(wq_ref[...] * war, axis=1)
    cl = jnp.sum(bk_ref[...] * wal, axis=1, keepdims=True) + bal_ref[...]
    cr = jnp.sum(bq_ref[...] * war, axis=1, keepdims=True) + bar_ref[...]
    col = lax.broadcasted_iota(jnp.int32, (D_IN, D_IN), 1)
    m2 = (jnp.where(col == 0, wkal[:, None], 0.0)
          + jnp.where(col == 1, wqar[:, None], 0.0))
    hz = jnp.dot(z, m2, preferred_element_type=f32)
    hd = jnp.dot(d, m2, preferred_element_type=f32)
    hl = hz[:, 0:1] + cl
    hr = hz[:, 1:2] + cr
    ha = hd[:, 0:1] + cl

    def elu(v):
        return jnp.where(v > 0, v, jnp.exp(v) - 1.0)

    a0 = elu(hl + hr)
    a1 = elu(ha + hr)
    mx = jnp.maximum(a0, a1)
    e0 = jnp.exp(a0 - mx)
    e1 = jnp.exp(a1 - mx)
    inv = 1.0 / (e0 + e1)
    rst = z * (e0 * inv) + d * (e1 * inv)
    out_ref[...] = (jnp.dot(rst, wcls_ref[...], preferred_element_type=f32)
                    + bcls_ref[...])


def _full(shape):
    return pl.BlockSpec(shape, lambda i: (0,) * len(shape))


_tc_epilogue = pl.pallas_call(
    _tc_body,
    grid=(_GRID,),
    in_specs=[
        pl.BlockSpec((BLK, D_IN), lambda i: (i, 0)),
        pl.BlockSpec((NGRP, BLK, GW), lambda i: (0, i, 0)),
        pl.BlockSpec((2, BLK, 16), lambda i: (0, i, 0)),
        _full((D_IN, D_IN)), _full((1, D_IN)),
        _full((D_IN, D_IN)), _full((1, D_IN)),
        _full((D_IN, D_IN)), _full((1, D_IN)),
        _full((D_IN, D_IN)), _full((1, D_IN)),
        _full((1, D_IN)), _full((1, 1)),
        _full((1, D_IN)), _full((1, 1)),
        _full((D_IN, 16)), _full((1, 16)),
    ],
    out_specs=pl.BlockSpec((BLK, 16), lambda i: (i, 0)),
    out_shape=jax.ShapeDtypeStruct((N_NODE, 16), jnp.float32),
)


def kernel(x_paper, x_author, edge_index_writes, edge_index_written_by,
           Wself_paper, bself_paper, Wself_author, bself_author,
           Wq_paper, bq_paper, Wk_paper, bk_paper,
           Wq_author, bq_author, Wk_author, bk_author,
           Wal_paper, bal_paper, Wal_author, bal_author,
           War_paper, bar_paper, War_author, bar_author,
           Wconv_writes, bconv_writes, Wconv_written_by, bconv_written_by,
           Wcls, bcls):
    e = edge_index_writes.shape[1]
    src = edge_index_writes[0]
    dst = edge_index_writes[1]
    npad = EP - e
    pad_src = jnp.arange(npad, dtype=jnp.int32) % N_NODE
    pad_dst = N_NODE + jnp.arange(npad, dtype=jnp.int32) % (NP - N_NODE)
    srcp = jnp.concatenate([src.astype(jnp.int32), pad_src])
    dstp = jnp.concatenate([dst.astype(jnp.int32), pad_dst])
    src2 = (srcp[None, :] * NGRP
            + jnp.arange(NGRP, dtype=jnp.int32)[:, None]).reshape(
                NGRP, NROW_E, 128)
    x2s = x_author.astype(jnp.bfloat16).reshape(NGRP * N_NODE, GW)

    dstr = dstp.reshape(NROW_E, 128)
    acc2 = _sc_scatter()(x2s, src2, dstr)
    deg2 = _sc_degree()(dstr)

    logits = _tc_epilogue(
        x_paper, acc2, deg2,
        Wself_paper, bself_paper.reshape(1, D_IN),
        Wconv_writes, bconv_writes.reshape(1, D_IN),
        Wq_paper, bq_paper.reshape(1, D_IN),
        Wk_paper, bk_paper.reshape(1, D_IN),
        Wal_paper.reshape(1, D_IN), bal_paper.reshape(1, 1),
        War_paper.reshape(1, D_IN), bar_paper.reshape(1, 1),
        Wcls, bcls.reshape(1, 16),
    )
    return logits
